# Initial kernel scaffold; baseline (speedup 1.0000x reference)
import jax
import jax.numpy as jnp
from jax.experimental import pallas as pl


def kernel(x, params, edge_index):
    raise NotImplementedError("write your pallas kernel here")


# speedup vs baseline: 1.0005x; 1.0005x over previous
import numpy as np
import jax
import jax.numpy as jnp
from jax.experimental import pallas as pl
from jax.experimental.pallas import tpu as pltpu

_N = 10000
_DIN = 256
_HID = 128
_H = 8
_DH = 128
_KP = 32
_PDIM = 32


def _mk_pairs(kp=_KP, pdim=_PDIM):
    ii, jj = [], []
    done = False
    for i in range(kp):
        for j in range(i + 1, kp):
            ii.append(i)
            jj.append(j)
            if len(ii) >= pdim:
                done = True
                break
        if done:
            break
    return np.array(ii), np.array(jj)


_PI, _PJ = _mk_pairs()


def _layer_norm(x, g, b, eps=1e-5):
    m = x.mean(-1, keepdims=True)
    v = ((x - m) ** 2).mean(-1, keepdims=True)
    return (x - m) / jnp.sqrt(v + eps) * g + b


def _gat_layer(x, edge_index, W1, b1, W2, b2, a):
    n = x.shape[0]
    h, d = a.shape
    h1 = (x @ W1.T + b1).reshape(n, h, d)
    h2 = (x @ W2.T + b2).reshape(n, h, d)
    hp = h1 * h2
    src = edge_index[0]
    dst = edge_index[1]
    attn_in = jax.nn.elu(h1[src] + h2[dst] + hp[src] * hp[dst])
    logits = jnp.einsum('hd,ehd->eh', a, attn_in) / np.sqrt(d)
    attn = jax.nn.softmax(logits, axis=1)
    msg = attn[:, :, None] * h1[src]
    out = jax.ops.segment_sum(msg, dst, num_segments=n)
    return out.reshape(n, -1)


def _final_proj_body(h_ref, w_ref, b_ref, o_ref):
    o_ref[...] = (
        jnp.dot(h_ref[...], w_ref[...], preferred_element_type=jnp.float32)
        + b_ref[...]
    )


def _final_proj(h, Wo, bo):
    n, hd = h.shape
    out_d = Wo.shape[0]
    blk = 2000
    return pl.pallas_call(
        _final_proj_body,
        grid=(n // blk,),
        in_specs=[
            pl.BlockSpec((blk, hd), lambda i: (i, 0)),
            pl.BlockSpec((hd, out_d), lambda i: (0, 0)),
            pl.BlockSpec((1, out_d), lambda i: (0, 0)),
        ],
        out_specs=pl.BlockSpec((blk, out_d), lambda i: (i, 0)),
        out_shape=jax.ShapeDtypeStruct((n, out_d), jnp.float32),
    )(h, Wo.T, bo.reshape(1, out_d))


def kernel(x, params, edge_index):
    p = params
    z = x @ p['Wp'].T + p['bp']
    phi = jnp.tanh(z)
    qc = jnp.cos(np.pi * phi)
    qs = jnp.sin(np.pi * phi)
    zv = jnp.var(z, axis=0, ddof=1)
    _, topk = jax.lax.top_k(zv, _KP)
    phi_i = jnp.take(phi, topk[_PI], axis=1)
    phi_j = jnp.take(phi, topk[_PJ], axis=1)
    qp = jnp.cos(phi_i - phi_j)
    Q = jnp.concatenate([qc, qs, qp], axis=1)
    Q = Q @ p['Wc'].T + p['bc']
    Q = _layer_norm(Q, p['lnq_g'], p['lnq_b']) * p['alpha']
    xc = Q @ p['Wlc'].T + p['blc']
    xc = jax.nn.elu(_layer_norm(xc, p['lnc_g'], p['lnc_b']))
    hcur = xc
    for li in range(2):
        res = hcur
        hcur = _gat_layer(hcur, edge_index,
                          p['g%d_W1' % li], p['g%d_b1' % li],
                          p['g%d_W2' % li], p['g%d_b2' % li],
                          p['g%d_a' % li])
        hcur = _layer_norm(hcur, p['ln%d_g' % li], p['ln%d_b' % li])
        hcur = jax.nn.elu(hcur)
        if res.shape == hcur.shape:
            hcur = hcur + res
    return _final_proj(hcur, p['Wo'], p['bo'])


# speedup vs baseline: 4.1600x; 4.1581x over previous
import functools

import numpy as np
import jax
import jax.numpy as jnp
from jax.experimental import pallas as pl
from jax.experimental.pallas import tpu as pltpu
from jax.experimental.pallas import tpu_sc as plsc

_N = 10000
_E = 160000
_DIN = 256
_HID = 128
_H = 8
_DH = 128
_HD = _H * _DH
_KP = 32
_PDIM = 32


def _mk_pairs(kp=_KP, pdim=_PDIM):
    ii, jj = [], []
    done = False
    for i in range(kp):
        for j in range(i + 1, kp):
            ii.append(i)
            jj.append(j)
            if len(ii) >= pdim:
                done = True
                break
        if done:
            break
    return np.array(ii), np.array(jj)


_PI, _PJ = _mk_pairs()


def _layer_norm(x, g, b, eps=1e-5):
    m = x.mean(-1, keepdims=True)
    v = ((x - m) ** 2).mean(-1, keepdims=True)
    return (x - m) / jnp.sqrt(v + eps) * g + b


def _sc_gather_call(Ts, Td, src, dst):
    info = plsc.get_sparse_core_info()
    NC, NS = info.num_cores, info.num_subcores
    NW = NC * NS
    per_w = _E // NW
    C = 8
    n_it = per_w // C
    W2 = 2 * _HD
    mesh = plsc.VectorSubcoreMesh(core_axis_name="c", subcore_axis_name="s")

    @functools.partial(
        pl.kernel,
        mesh=mesh,
        out_type=[
            jax.ShapeDtypeStruct((_E, W2), jnp.float32),
            jax.ShapeDtypeStruct((_E, W2), jnp.float32),
        ],
        scratch_types=[
            pltpu.VMEM((C,), jnp.int32),
            pltpu.VMEM((C,), jnp.int32),
            pltpu.VMEM((C, W2), jnp.float32),
            pltpu.VMEM((C, W2), jnp.float32),
            pltpu.SemaphoreType.DMA,
        ],
    )
    def k(ts_hbm, td_hbm, src_hbm, dst_hbm, gs_hbm, gd_hbm,
          sidx, didx, bufs, bufd, sem):
        wid = jax.lax.axis_index("s") * NC + jax.lax.axis_index("c")
        base = wid * per_w

        def body(i, carry):
            e0 = base + i * C
            pltpu.sync_copy(src_hbm.at[pl.ds(e0, C)], sidx)
            pltpu.sync_copy(dst_hbm.at[pl.ds(e0, C)], didx)
            d1 = pltpu.async_copy(ts_hbm.at[sidx], bufs, sem)
            d2 = pltpu.async_copy(td_hbm.at[didx], bufd, sem)
            d1.wait()
            d2.wait()
            pltpu.sync_copy(bufs, gs_hbm.at[pl.ds(e0, C)])
            pltpu.sync_copy(bufd, gd_hbm.at[pl.ds(e0, C)])
            return carry

        jax.lax.fori_loop(0, n_it, body, 0)

    return k(Ts, Td, src, dst)


def _attn_msg_body(gs_ref, gd_ref, a_ref, msg_ref, *, eb):
    h1s = gs_ref[:, :_HD]
    hps = gs_ref[:, _HD:]
    h2d = gd_ref[:, :_HD]
    hpd = gd_ref[:, _HD:]
    t = h1s + h2d + hps * hpd
    t = jnp.where(t > 0, t, jnp.exp(t) - 1.0)
    w = (t * a_ref[...]).reshape(eb, _H, _DH)
    logits = jnp.sum(w, axis=-1) * np.float32(1.0 / np.sqrt(_DH))
    m = jnp.max(logits, axis=1, keepdims=True)
    ex = jnp.exp(logits - m)
    attn = ex / jnp.sum(ex, axis=1, keepdims=True)
    msg = attn[:, :, None] * h1s.reshape(eb, _H, _DH)
    msg_ref[...] = msg.reshape(eb, _HD)


def _attn_msg_call(Gs, Gd, a_flat):
    eb = 320
    W2 = 2 * _HD
    return pl.pallas_call(
        functools.partial(_attn_msg_body, eb=eb),
        grid=(_E // eb,),
        in_specs=[
            pl.BlockSpec((eb, W2), lambda i: (i, 0)),
            pl.BlockSpec((eb, W2), lambda i: (i, 0)),
            pl.BlockSpec((1, _HD), lambda i: (0, 0)),
        ],
        out_specs=pl.BlockSpec((eb, _HD), lambda i: (i, 0)),
        out_shape=jax.ShapeDtypeStruct((_E, _HD), jnp.float32),
    )(Gs, Gd, a_flat)


def _sc_scatter_call(msg, dst, zrows):
    info = plsc.get_sparse_core_info()
    NC, NS = info.num_cores, info.num_subcores
    per_t = _E // NS
    C2 = 80
    n_it = per_t // C2
    NPAD = 10240
    zpt = NPAD // NS
    opt = 624
    hpc = _H // NC
    mesh = plsc.VectorSubcoreMesh(core_axis_name="c", subcore_axis_name="s")

    @functools.partial(
        pl.kernel,
        mesh=mesh,
        out_type=jax.ShapeDtypeStruct((_N, _HD), jnp.float32),
        scratch_types=[
            pltpu.VMEM((C2,), jnp.int32),
            pltpu.VMEM((C2, _DH), jnp.float32),
            pltpu.VMEM_SHARED((NPAD, _DH), jnp.float32),
            pltpu.SemaphoreType.DMA,
        ],
    )
    def k(msg_hbm, dst_hbm, z_hbm, out_hbm, didx, mbuf, acc_sh, sem):
        c = jax.lax.axis_index("c")
        s = jax.lax.axis_index("s")
        for hh in range(hpc):
            h = c * hpc + hh
            col = h * _DH
            pltpu.sync_copy(z_hbm, acc_sh.at[pl.ds(s * zpt, zpt)])
            plsc.subcore_barrier()

            def body(i, carry):
                e0 = s * per_t + i * C2
                pltpu.sync_copy(dst_hbm.at[pl.ds(e0, C2)], didx)
                pltpu.sync_copy(msg_hbm.at[pl.ds(e0, C2), pl.ds(col, _DH)],
                                mbuf)
                pltpu.sync_copy(mbuf, acc_sh.at[didx], add=True)
                return carry

            jax.lax.fori_loop(0, n_it, body, 0)
            plsc.subcore_barrier()
            pltpu.sync_copy(
                acc_sh.at[pl.ds(s * opt, opt)],
                out_hbm.at[pl.ds(s * opt, opt), pl.ds(col, _DH)])

            @pl.when(s == NS - 1)
            def _():
                pltpu.sync_copy(
                    acc_sh.at[pl.ds(NS * opt, _N - NS * opt)],
                    out_hbm.at[pl.ds(NS * opt, _N - NS * opt),
                               pl.ds(col, _DH)])

            plsc.subcore_barrier()

    return k(msg, dst, zrows)


def _gat_layer_sc(hcur, src, dst, zrows, W1, b1, W2, b2, a):
    h1 = hcur @ W1.T + b1
    h2 = hcur @ W2.T + b2
    hp = h1 * h2
    Ts = jnp.concatenate([h1, hp], axis=1)
    Td = jnp.concatenate([h2, hp], axis=1)
    Gs, Gd = _sc_gather_call(Ts, Td, src, dst)
    msg = _attn_msg_call(Gs, Gd, a.reshape(1, _HD))
    return _sc_scatter_call(msg, dst, zrows)


def _final_proj_body(h_ref, w_ref, b_ref, o_ref):
    o_ref[...] = (
        jnp.dot(h_ref[...], w_ref[...], preferred_element_type=jnp.float32)
        + b_ref[...]
    )


def _final_proj(h, Wo, bo):
    n, hd = h.shape
    out_d = Wo.shape[0]
    blk = 2000
    return pl.pallas_call(
        _final_proj_body,
        grid=(n // blk,),
        in_specs=[
            pl.BlockSpec((blk, hd), lambda i: (i, 0)),
            pl.BlockSpec((hd, out_d), lambda i: (0, 0)),
            pl.BlockSpec((1, out_d), lambda i: (0, 0)),
        ],
        out_specs=pl.BlockSpec((blk, out_d), lambda i: (i, 0)),
        out_shape=jax.ShapeDtypeStruct((n, out_d), jnp.float32),
    )(h, Wo.T, bo.reshape(1, out_d))


def kernel(x, params, edge_index):
    p = params
    src = edge_index[0]
    dst = edge_index[1]
    zrows = jnp.zeros((10240 // 16, _DH), jnp.float32)
    z = x @ p['Wp'].T + p['bp']
    phi = jnp.tanh(z)
    qc = jnp.cos(np.pi * phi)
    qs = jnp.sin(np.pi * phi)
    zv = jnp.var(z, axis=0, ddof=1)
    _, topk = jax.lax.top_k(zv, _KP)
    phi_i = jnp.take(phi, topk[_PI], axis=1)
    phi_j = jnp.take(phi, topk[_PJ], axis=1)
    qp = jnp.cos(phi_i - phi_j)
    Q = jnp.concatenate([qc, qs, qp], axis=1)
    Q = Q @ p['Wc'].T + p['bc']
    Q = _layer_norm(Q, p['lnq_g'], p['lnq_b']) * p['alpha']
    xc = Q @ p['Wlc'].T + p['blc']
    xc = jax.nn.elu(_layer_norm(xc, p['lnc_g'], p['lnc_b']))
    hcur = xc
    for li in range(2):
        res = hcur
        hcur = _gat_layer_sc(hcur, src, dst, zrows,
                             p['g%d_W1' % li], p['g%d_b1' % li],
                             p['g%d_W2' % li], p['g%d_b2' % li],
                             p['g%d_a' % li])
        hcur = _layer_norm(hcur, p['ln%d_g' % li], p['ln%d_b' % li])
        hcur = jax.nn.elu(hcur)
        if res.shape == hcur.shape:
            hcur = hcur + res
    return _final_proj(hcur, p['Wo'], p['bo'])


# speedup vs baseline: 4.2925x; 1.0318x over previous
import functools

import numpy as np
import jax
import jax.numpy as jnp
from jax.experimental import pallas as pl
from jax.experimental.pallas import tpu as pltpu
from jax.experimental.pallas import tpu_sc as plsc

_N = 10000
_E = 160000
_DIN = 256
_HID = 128
_H = 8
_DH = 128
_HD = _H * _DH
_KP = 32
_PDIM = 32


def _mk_pairs(kp=_KP, pdim=_PDIM):
    ii, jj = [], []
    done = False
    for i in range(kp):
        for j in range(i + 1, kp):
            ii.append(i)
            jj.append(j)
            if len(ii) >= pdim:
                done = True
                break
        if done:
            break
    return np.array(ii), np.array(jj)


_PI, _PJ = _mk_pairs()


def _layer_norm(x, g, b, eps=1e-5):
    m = x.mean(-1, keepdims=True)
    v = ((x - m) ** 2).mean(-1, keepdims=True)
    return (x - m) / jnp.sqrt(v + eps) * g + b


def _sc_gather_call(Ts, Td, src, dst):
    info = plsc.get_sparse_core_info()
    NC, NS = info.num_cores, info.num_subcores
    NW = NC * NS
    per_w = _E // NW
    C = 8
    n_it = per_w // C
    W2 = 2 * _HD
    mesh = plsc.VectorSubcoreMesh(core_axis_name="c", subcore_axis_name="s")

    @functools.partial(
        pl.kernel,
        mesh=mesh,
        out_type=[
            jax.ShapeDtypeStruct((_E, W2), jnp.float32),
            jax.ShapeDtypeStruct((_E, W2), jnp.float32),
        ],
        scratch_types=[
            pltpu.VMEM((C,), jnp.int32),
            pltpu.VMEM((C,), jnp.int32),
            pltpu.VMEM((C, W2), jnp.float32),
            pltpu.VMEM((C, W2), jnp.float32),
            pltpu.SemaphoreType.DMA,
        ],
    )
    def k(ts_hbm, td_hbm, src_hbm, dst_hbm, gs_hbm, gd_hbm,
          sidx, didx, bufs, bufd, sem):
        wid = jax.lax.axis_index("s") * NC + jax.lax.axis_index("c")
        base = wid * per_w

        def body(i, carry):
            e0 = base + i * C
            pltpu.sync_copy(src_hbm.at[pl.ds(e0, C)], sidx)
            pltpu.sync_copy(dst_hbm.at[pl.ds(e0, C)], didx)
            d1 = pltpu.async_copy(ts_hbm.at[sidx], bufs, sem)
            d2 = pltpu.async_copy(td_hbm.at[didx], bufd, sem)
            d1.wait()
            d2.wait()
            pltpu.sync_copy(bufs, gs_hbm.at[pl.ds(e0, C)])
            pltpu.sync_copy(bufd, gd_hbm.at[pl.ds(e0, C)])
            return carry

        jax.lax.fori_loop(0, n_it, body, 0)

    return k(Ts, Td, src, dst)


def _attn_msg_body(gs_ref, gd_ref, a_ref, msg_ref, *, eb):
    h1s = gs_ref[:, :_HD]
    hps = gs_ref[:, _HD:]
    h2d = gd_ref[:, :_HD]
    hpd = gd_ref[:, _HD:]
    t = h1s + h2d + hps * hpd
    t = jnp.where(t > 0, t, jnp.exp(t) - 1.0)
    w = (t * a_ref[...]).reshape(eb, _H, _DH)
    logits = jnp.sum(w, axis=-1) * np.float32(1.0 / np.sqrt(_DH))
    m = jnp.max(logits, axis=1, keepdims=True)
    ex = jnp.exp(logits - m)
    attn = ex / jnp.sum(ex, axis=1, keepdims=True)
    msg = attn[:, :, None] * h1s.reshape(eb, _H, _DH)
    msg_ref[...] = msg.reshape(eb, _HD)


def _attn_msg_call(Gs, Gd, a_flat):
    eb = 320
    W2 = 2 * _HD
    return pl.pallas_call(
        functools.partial(_attn_msg_body, eb=eb),
        grid=(_E // eb,),
        in_specs=[
            pl.BlockSpec((eb, W2), lambda i: (i, 0)),
            pl.BlockSpec((eb, W2), lambda i: (i, 0)),
            pl.BlockSpec((1, _HD), lambda i: (0, 0)),
        ],
        out_specs=pl.BlockSpec((eb, _HD), lambda i: (i, 0)),
        out_shape=jax.ShapeDtypeStruct((_E, _HD), jnp.float32),
    )(Gs, Gd, a_flat)


def _sc_scatter_call(msg, dst, zrows):
    info = plsc.get_sparse_core_info()
    NC, NS = info.num_cores, info.num_subcores
    per_t = _E // NS
    C2 = 80
    n_it = per_t // C2
    NPAD = 10240
    zpt = NPAD // NS
    opt = 624
    hpc = _H // NC
    mesh = plsc.VectorSubcoreMesh(core_axis_name="c", subcore_axis_name="s")

    @functools.partial(
        pl.kernel,
        mesh=mesh,
        out_type=jax.ShapeDtypeStruct((_N, _HD), jnp.float32),
        scratch_types=[
            pltpu.VMEM((C2,), jnp.int32),
            pltpu.VMEM((C2, _DH), jnp.float32),
            pltpu.VMEM_SHARED((NPAD, _DH), jnp.float32),
            pltpu.SemaphoreType.DMA,
        ],
    )
    def k(msg_hbm, dst_hbm, z_hbm, out_hbm, didx, mbuf, acc_sh, sem):
        c = jax.lax.axis_index("c")
        s = jax.lax.axis_index("s")
        for hh in range(hpc):
            h = c * hpc + hh
            col = h * _DH
            pltpu.sync_copy(z_hbm, acc_sh.at[pl.ds(s * zpt, zpt)])
            plsc.subcore_barrier()

            def body(i, carry):
                e0 = s * per_t + i * C2
                pltpu.sync_copy(dst_hbm.at[pl.ds(e0, C2)], didx)
                pltpu.sync_copy(msg_hbm.at[pl.ds(e0, C2), pl.ds(col, _DH)],
                                mbuf)
                pltpu.sync_copy(mbuf, acc_sh.at[didx], add=True)
                return carry

            jax.lax.fori_loop(0, n_it, body, 0)
            plsc.subcore_barrier()
            pltpu.sync_copy(
                acc_sh.at[pl.ds(s * opt, opt)],
                out_hbm.at[pl.ds(s * opt, opt), pl.ds(col, _DH)])

            @pl.when(s == NS - 1)
            def _():
                pltpu.sync_copy(
                    acc_sh.at[pl.ds(NS * opt, _N - NS * opt)],
                    out_hbm.at[pl.ds(NS * opt, _N - NS * opt),
                               pl.ds(col, _DH)])

            plsc.subcore_barrier()

    return k(msg, dst, zrows)


def _gat_edge_stage(Ts, Td, src, dst, zrows, a):
    Gs, Gd = _sc_gather_call(Ts, Td, src, dst)
    msg = _attn_msg_call(Gs, Gd, a.reshape(1, _HD))
    return _sc_scatter_call(msg, dst, zrows)


def _elu(v):
    return jnp.where(v > 0, v, jnp.exp(v) - 1.0)


def _front_stats_body(x_ref, w_ref, b_ref, phi_ref, s_ref, ss_ref):
    z = jnp.dot(x_ref[...], w_ref[...],
                preferred_element_type=jnp.float32) + b_ref[...]
    phi_ref[...] = jnp.tanh(z)

    @pl.when(pl.program_id(0) == 0)
    def _():
        s_ref[...] = jnp.zeros_like(s_ref)
        ss_ref[...] = jnp.zeros_like(ss_ref)

    s_ref[...] += jnp.sum(z, axis=0, keepdims=True)
    ss_ref[...] += jnp.sum(z * z, axis=0, keepdims=True)


def _front_stats(x, Wp, bp):
    blk = 2000
    return pl.pallas_call(
        _front_stats_body,
        grid=(_N // blk,),
        in_specs=[
            pl.BlockSpec((blk, _DIN), lambda i: (i, 0)),
            pl.BlockSpec((_DIN, _DIN), lambda i: (0, 0)),
            pl.BlockSpec((1, _DIN), lambda i: (0, 0)),
        ],
        out_specs=[
            pl.BlockSpec((blk, _DIN), lambda i: (i, 0)),
            pl.BlockSpec((1, _DIN), lambda i: (0, 0)),
            pl.BlockSpec((1, _DIN), lambda i: (0, 0)),
        ],
        out_shape=[
            jax.ShapeDtypeStruct((_N, _DIN), jnp.float32),
            jax.ShapeDtypeStruct((1, _DIN), jnp.float32),
            jax.ShapeDtypeStruct((1, _DIN), jnp.float32),
        ],
    )(x, Wp.T, bp.reshape(1, _DIN))


def _front_feat_body(phi_ref, si_ref, sj_ref, wa_ref, wb_ref, wc_ref,
                     bc_ref, lg_ref, lb_ref, al_ref, wl_ref, blc_ref,
                     cg_ref, cb_ref, xc_ref):
    phi = phi_ref[...]
    qc = jnp.cos(np.float32(np.pi) * phi)
    qs = jnp.sin(np.float32(np.pi) * phi)
    phi_i = jnp.dot(phi, si_ref[...], preferred_element_type=jnp.float32)
    phi_j = jnp.dot(phi, sj_ref[...], preferred_element_type=jnp.float32)
    qp = jnp.cos(phi_i - phi_j)
    Q = (jnp.dot(qc, wa_ref[...], preferred_element_type=jnp.float32)
         + jnp.dot(qs, wb_ref[...], preferred_element_type=jnp.float32)
         + jnp.dot(qp, wc_ref[...], preferred_element_type=jnp.float32)
         + bc_ref[...])
    Q = _layer_norm(Q, lg_ref[...], lb_ref[...]) * al_ref[0, 0]
    xc = jnp.dot(Q, wl_ref[...],
                 preferred_element_type=jnp.float32) + blc_ref[...]
    xc_ref[...] = _elu(_layer_norm(xc, cg_ref[...], cb_ref[...]))


def _front_feat(phi, Si, Sj, Wc, bc, lnq_g, lnq_b, alpha, Wlc, blc,
                lnc_g, lnc_b):
    blk = 2000
    WcT = Wc.T
    return pl.pallas_call(
        _front_feat_body,
        grid=(_N // blk,),
        in_specs=[
            pl.BlockSpec((blk, _DIN), lambda i: (i, 0)),
            pl.BlockSpec((_DIN, _PDIM), lambda i: (0, 0)),
            pl.BlockSpec((_DIN, _PDIM), lambda i: (0, 0)),
            pl.BlockSpec((_DIN, _DIN), lambda i: (0, 0)),
            pl.BlockSpec((_DIN, _DIN), lambda i: (0, 0)),
            pl.BlockSpec((_PDIM, _DIN), lambda i: (0, 0)),
            pl.BlockSpec((1, _DIN), lambda i: (0, 0)),
            pl.BlockSpec((1, _DIN), lambda i: (0, 0)),
            pl.BlockSpec((1, _DIN), lambda i: (0, 0)),
            pl.BlockSpec((1, 1), lambda i: (0, 0)),
            pl.BlockSpec((_DIN, _HID), lambda i: (0, 0)),
            pl.BlockSpec((1, _HID), lambda i: (0, 0)),
            pl.BlockSpec((1, _HID), lambda i: (0, 0)),
            pl.BlockSpec((1, _HID), lambda i: (0, 0)),
        ],
        out_specs=pl.BlockSpec((blk, _HID), lambda i: (i, 0)),
        out_shape=jax.ShapeDtypeStruct((_N, _HID), jnp.float32),
    )(phi, Si, Sj, WcT[:_DIN], WcT[_DIN:2 * _DIN], WcT[2 * _DIN:],
      bc.reshape(1, _DIN), lnq_g.reshape(1, _DIN), lnq_b.reshape(1, _DIN),
      alpha.reshape(1, 1), Wlc.T, blc.reshape(1, _HID),
      lnc_g.reshape(1, _HID), lnc_b.reshape(1, _HID))


def _proj_body(h_ref, w1_ref, b1_ref, w2_ref, b2_ref, g_ref, b_ref,
               ts_ref, td_ref, hout_ref, *, pre_ln):
    hin = h_ref[...]
    if pre_ln:
        hin = _elu(_layer_norm(hin, g_ref[...], b_ref[...]))
        hout_ref[...] = hin
    h1 = jnp.dot(hin, w1_ref[...],
                 preferred_element_type=jnp.float32) + b1_ref[...]
    h2 = jnp.dot(hin, w2_ref[...],
                 preferred_element_type=jnp.float32) + b2_ref[...]
    hp = h1 * h2
    ts_ref[:, :_HD] = h1
    ts_ref[:, _HD:] = hp
    td_ref[:, :_HD] = h2
    td_ref[:, _HD:] = hp


def _layer_proj(hin, W1, b1, W2, b2, ln_g, ln_b, pre_ln):
    blk = 400
    k = hin.shape[1]
    outs = [
        jax.ShapeDtypeStruct((_N, 2 * _HD), jnp.float32),
        jax.ShapeDtypeStruct((_N, 2 * _HD), jnp.float32),
        jax.ShapeDtypeStruct((_N, k), jnp.float32),
    ]
    res = pl.pallas_call(
        functools.partial(_proj_body, pre_ln=pre_ln),
        grid=(_N // blk,),
        in_specs=[
            pl.BlockSpec((blk, k), lambda i: (i, 0)),
            pl.BlockSpec((k, _HD), lambda i: (0, 0)),
            pl.BlockSpec((1, _HD), lambda i: (0, 0)),
            pl.BlockSpec((k, _HD), lambda i: (0, 0)),
            pl.BlockSpec((1, _HD), lambda i: (0, 0)),
            pl.BlockSpec((1, k), lambda i: (0, 0)),
            pl.BlockSpec((1, k), lambda i: (0, 0)),
        ],
        out_specs=[
            pl.BlockSpec((blk, 2 * _HD), lambda i: (i, 0)),
            pl.BlockSpec((blk, 2 * _HD), lambda i: (i, 0)),
            pl.BlockSpec((blk, k), lambda i: (i, 0)),
        ],
        out_shape=outs,
    )(hin, W1.T, b1.reshape(1, _HD), W2.T, b2.reshape(1, _HD),
      ln_g.reshape(1, k), ln_b.reshape(1, k))
    return res[0], res[1], res[2]


def _final_body(o1_ref, r_ref, g_ref, b_ref, w_ref, bo_ref, y_ref):
    h = _elu(_layer_norm(o1_ref[...], g_ref[...], b_ref[...])) + r_ref[...]
    y_ref[...] = jnp.dot(h, w_ref[...],
                         preferred_element_type=jnp.float32) + bo_ref[...]


def _final_proj(out1, hres, ln_g, ln_b, Wo, bo):
    blk = 1000
    out_d = Wo.shape[0]
    return pl.pallas_call(
        _final_body,
        grid=(_N // blk,),
        in_specs=[
            pl.BlockSpec((blk, _HD), lambda i: (i, 0)),
            pl.BlockSpec((blk, _HD), lambda i: (i, 0)),
            pl.BlockSpec((1, _HD), lambda i: (0, 0)),
            pl.BlockSpec((1, _HD), lambda i: (0, 0)),
            pl.BlockSpec((_HD, out_d), lambda i: (0, 0)),
            pl.BlockSpec((1, out_d), lambda i: (0, 0)),
        ],
        out_specs=pl.BlockSpec((blk, out_d), lambda i: (i, 0)),
        out_shape=jax.ShapeDtypeStruct((_N, out_d), jnp.float32),
    )(out1, hres, ln_g.reshape(1, _HD), ln_b.reshape(1, _HD), Wo.T,
      bo.reshape(1, out_d))


def kernel(x, params, edge_index):
    p = params
    src = edge_index[0]
    dst = edge_index[1]
    zrows = jnp.zeros((10240 // 16, _DH), jnp.float32)

    phi, ssum, ssq = _front_stats(x, p['Wp'], p['bp'])
    zv = ((ssq - ssum * ssum / _N) / (_N - 1)).reshape(_DIN)
    _, topk = jax.lax.top_k(zv, _KP)
    ar = jnp.arange(_DIN, dtype=jnp.int32)
    Si = (ar[:, None] == topk[_PI][None, :]).astype(jnp.float32)
    Sj = (ar[:, None] == topk[_PJ][None, :]).astype(jnp.float32)
    xc = _front_feat(phi, Si, Sj, p['Wc'], p['bc'], p['lnq_g'], p['lnq_b'],
                     p['alpha'], p['Wlc'], p['blc'], p['lnc_g'], p['lnc_b'])

    Ts0, Td0, _unused = _layer_proj(xc, p['g0_W1'], p['g0_b1'],
                                    p['g0_W2'], p['g0_b2'],
                                    p['lnc_g'], p['lnc_b'], pre_ln=False)
    out0 = _gat_edge_stage(Ts0, Td0, src, dst, zrows, p['g0_a'])

    Ts1, Td1, hres = _layer_proj(out0, p['g1_W1'], p['g1_b1'],
                                 p['g1_W2'], p['g1_b2'],
                                 p['ln0_g'], p['ln0_b'], pre_ln=True)
    out1 = _gat_edge_stage(Ts1, Td1, src, dst, zrows, p['g1_a'])

    return _final_proj(out1, hres, p['ln1_g'], p['ln1_b'], p['Wo'], p['bo'])


# speedup vs baseline: 5.1473x; 1.1991x over previous
import functools

import numpy as np
import jax
import jax.numpy as jnp
from jax.experimental import pallas as pl
from jax.experimental.pallas import tpu as pltpu
from jax.experimental.pallas import tpu_sc as plsc

_N = 10000
_E = 160000
_DIN = 256
_HID = 128
_H = 8
_DH = 128
_HD = _H * _DH
_KP = 32
_PDIM = 32


def _mk_pairs(kp=_KP, pdim=_PDIM):
    ii, jj = [], []
    done = False
    for i in range(kp):
        for j in range(i + 1, kp):
            ii.append(i)
            jj.append(j)
            if len(ii) >= pdim:
                done = True
                break
        if done:
            break
    return np.array(ii), np.array(jj)


_PI, _PJ = _mk_pairs()


def _layer_norm(x, g, b, eps=1e-5):
    m = x.mean(-1, keepdims=True)
    v = ((x - m) ** 2).mean(-1, keepdims=True)
    return (x - m) / jnp.sqrt(v + eps) * g + b


def _sc_gather_call(Ts, Td, src, dst):
    info = plsc.get_sparse_core_info()
    NC, NS = info.num_cores, info.num_subcores
    NW = NC * NS
    per_w = _E // NW
    C = 8
    n_it = per_w // C
    W2 = 2 * _HD
    mesh = plsc.VectorSubcoreMesh(core_axis_name="c", subcore_axis_name="s")

    @functools.partial(
        pl.kernel,
        mesh=mesh,
        out_type=[
            jax.ShapeDtypeStruct((_E, W2), jnp.float32),
            jax.ShapeDtypeStruct((_E, W2), jnp.float32),
        ],
        scratch_types=[
            pltpu.VMEM((2, C), jnp.int32),
            pltpu.VMEM((2, C), jnp.int32),
            pltpu.VMEM((2, C, W2), jnp.float32),
            pltpu.VMEM((2, C, W2), jnp.float32),
            pltpu.SemaphoreType.DMA,
            pltpu.SemaphoreType.DMA,
        ],
    )
    def k(ts_hbm, td_hbm, src_hbm, dst_hbm, gs_hbm, gd_hbm,
          sidx, didx, bufs, bufd, sem0, sem1):
        wid = jax.lax.axis_index("s") * NC + jax.lax.axis_index("c")
        base = wid * per_w
        sems = (sem0, sem1)

        def fire(j, b):
            e0 = base + j * C
            pltpu.sync_copy(src_hbm.at[pl.ds(e0, C)], sidx.at[b])
            pltpu.sync_copy(dst_hbm.at[pl.ds(e0, C)], didx.at[b])
            pltpu.async_copy(ts_hbm.at[sidx.at[b]], bufs.at[b], sems[b])
            pltpu.async_copy(td_hbm.at[didx.at[b]], bufd.at[b], sems[b])

        def consume(j, b):
            pltpu.make_async_copy(ts_hbm.at[sidx.at[b]], bufs.at[b],
                                  sems[b]).wait()
            pltpu.make_async_copy(td_hbm.at[didx.at[b]], bufd.at[b],
                                  sems[b]).wait()
            e0 = base + j * C
            pltpu.sync_copy(bufs.at[b], gs_hbm.at[pl.ds(e0, C)])
            pltpu.sync_copy(bufd.at[b], gd_hbm.at[pl.ds(e0, C)])

        fire(0, 0)

        def body(i, carry):
            j0 = 2 * i
            fire(j0 + 1, 1)
            consume(j0, 0)

            @pl.when(j0 + 2 < n_it)
            def _():
                fire(j0 + 2, 0)

            consume(j0 + 1, 1)
            return carry

        jax.lax.fori_loop(0, n_it // 2, body, 0)
        if n_it % 2 == 1:
            consume(n_it - 1, 0)

    return k(Ts, Td, src, dst)


def _attn_msg_body(gs_ref, gd_ref, a_ref, msg_ref, *, eb):
    h1s = gs_ref[:, :_HD]
    hps = gs_ref[:, _HD:]
    h2d = gd_ref[:, :_HD]
    hpd = gd_ref[:, _HD:]
    t = h1s + h2d + hps * hpd
    t = jnp.where(t > 0, t, jnp.exp(t) - 1.0)
    w = (t * a_ref[...]).reshape(eb, _H, _DH)
    logits = jnp.sum(w, axis=-1) * np.float32(1.0 / np.sqrt(_DH))
    m = jnp.max(logits, axis=1, keepdims=True)
    ex = jnp.exp(logits - m)
    attn = ex / jnp.sum(ex, axis=1, keepdims=True)
    msg = attn[:, :, None] * h1s.reshape(eb, _H, _DH)
    msg_ref[...] = msg.reshape(eb, _HD)


def _attn_msg_call(Gs, Gd, a_flat):
    eb = 320
    W2 = 2 * _HD
    return pl.pallas_call(
        functools.partial(_attn_msg_body, eb=eb),
        grid=(_E // eb,),
        in_specs=[
            pl.BlockSpec((eb, W2), lambda i: (i, 0)),
            pl.BlockSpec((eb, W2), lambda i: (i, 0)),
            pl.BlockSpec((1, _HD), lambda i: (0, 0)),
        ],
        out_specs=pl.BlockSpec((eb, _HD), lambda i: (i, 0)),
        out_shape=jax.ShapeDtypeStruct((_E, _HD), jnp.float32),
    )(Gs, Gd, a_flat)


def _sc_scatter_call(msg, dst, zrows):
    info = plsc.get_sparse_core_info()
    NC, NS = info.num_cores, info.num_subcores
    per_t = _E // NS
    C2 = 80
    n_it = per_t // C2
    NPAD = 10240
    zpt = NPAD // NS
    opt = 624
    hpc = _H // NC
    mesh = plsc.VectorSubcoreMesh(core_axis_name="c", subcore_axis_name="s")

    @functools.partial(
        pl.kernel,
        mesh=mesh,
        out_type=jax.ShapeDtypeStruct((_N, _HD), jnp.float32),
        scratch_types=[
            pltpu.VMEM((C2,), jnp.int32),
            pltpu.VMEM((C2, _DH), jnp.float32),
            pltpu.VMEM_SHARED((NPAD, _DH), jnp.float32),
            pltpu.SemaphoreType.DMA,
        ],
    )
    def k(msg_hbm, dst_hbm, z_hbm, out_hbm, didx, mbuf, acc_sh, sem):
        c = jax.lax.axis_index("c")
        s = jax.lax.axis_index("s")
        for hh in range(hpc):
            h = c * hpc + hh
            col = h * _DH
            pltpu.sync_copy(z_hbm, acc_sh.at[pl.ds(s * zpt, zpt)])
            plsc.subcore_barrier()

            def body(i, carry):
                e0 = s * per_t + i * C2
                pltpu.sync_copy(dst_hbm.at[pl.ds(e0, C2)], didx)
                pltpu.sync_copy(msg_hbm.at[pl.ds(e0, C2), pl.ds(col, _DH)],
                                mbuf)
                pltpu.sync_copy(mbuf, acc_sh.at[didx], add=True)
                return carry

            jax.lax.fori_loop(0, n_it, body, 0)
            plsc.subcore_barrier()
            pltpu.sync_copy(
                acc_sh.at[pl.ds(s * opt, opt)],
                out_hbm.at[pl.ds(s * opt, opt), pl.ds(col, _DH)])

            @pl.when(s == NS - 1)
            def _():
                pltpu.sync_copy(
                    acc_sh.at[pl.ds(NS * opt, _N - NS * opt)],
                    out_hbm.at[pl.ds(NS * opt, _N - NS * opt),
                               pl.ds(col, _DH)])

            plsc.subcore_barrier()

    return k(msg, dst, zrows)


def _gat_edge_stage(Ts, Td, src, dst, zrows, a):
    Gs, Gd = _sc_gather_call(Ts, Td, src, dst)
    msg = _attn_msg_call(Gs, Gd, a.reshape(1, _HD))
    return _sc_scatter_call(msg, dst, zrows)


def _elu(v):
    return jnp.where(v > 0, v, jnp.exp(v) - 1.0)


def _front_stats_body(x_ref, w_ref, b_ref, phi_ref, s_ref, ss_ref):
    z = jnp.dot(x_ref[...], w_ref[...],
                preferred_element_type=jnp.float32) + b_ref[...]
    phi_ref[...] = jnp.tanh(z)

    @pl.when(pl.program_id(0) == 0)
    def _():
        s_ref[...] = jnp.zeros_like(s_ref)
        ss_ref[...] = jnp.zeros_like(ss_ref)

    s_ref[...] += jnp.sum(z, axis=0, keepdims=True)
    ss_ref[...] += jnp.sum(z * z, axis=0, keepdims=True)


def _front_stats(x, Wp, bp):
    blk = 2000
    return pl.pallas_call(
        _front_stats_body,
        grid=(_N // blk,),
        in_specs=[
            pl.BlockSpec((blk, _DIN), lambda i: (i, 0)),
            pl.BlockSpec((_DIN, _DIN), lambda i: (0, 0)),
            pl.BlockSpec((1, _DIN), lambda i: (0, 0)),
        ],
        out_specs=[
            pl.BlockSpec((blk, _DIN), lambda i: (i, 0)),
            pl.BlockSpec((1, _DIN), lambda i: (0, 0)),
            pl.BlockSpec((1, _DIN), lambda i: (0, 0)),
        ],
        out_shape=[
            jax.ShapeDtypeStruct((_N, _DIN), jnp.float32),
            jax.ShapeDtypeStruct((1, _DIN), jnp.float32),
            jax.ShapeDtypeStruct((1, _DIN), jnp.float32),
        ],
    )(x, Wp.T, bp.reshape(1, _DIN))


def _front_feat_body(phi_ref, si_ref, sj_ref, wa_ref, wb_ref, wc_ref,
                     bc_ref, lg_ref, lb_ref, al_ref, wl_ref, blc_ref,
                     cg_ref, cb_ref, xc_ref):
    phi = phi_ref[...]
    qc = jnp.cos(np.float32(np.pi) * phi)
    qs = jnp.sin(np.float32(np.pi) * phi)
    phi_i = jnp.dot(phi, si_ref[...], preferred_element_type=jnp.float32)
    phi_j = jnp.dot(phi, sj_ref[...], preferred_element_type=jnp.float32)
    qp = jnp.cos(phi_i - phi_j)
    Q = (jnp.dot(qc, wa_ref[...], preferred_element_type=jnp.float32)
         + jnp.dot(qs, wb_ref[...], preferred_element_type=jnp.float32)
         + jnp.dot(qp, wc_ref[...], preferred_element_type=jnp.float32)
         + bc_ref[...])
    Q = _layer_norm(Q, lg_ref[...], lb_ref[...]) * al_ref[0, 0]
    xc = jnp.dot(Q, wl_ref[...],
                 preferred_element_type=jnp.float32) + blc_ref[...]
    xc_ref[...] = _elu(_layer_norm(xc, cg_ref[...], cb_ref[...]))


def _front_feat(phi, Si, Sj, Wc, bc, lnq_g, lnq_b, alpha, Wlc, blc,
                lnc_g, lnc_b):
    blk = 2000
    WcT = Wc.T
    return pl.pallas_call(
        _front_feat_body,
        grid=(_N // blk,),
        in_specs=[
            pl.BlockSpec((blk, _DIN), lambda i: (i, 0)),
            pl.BlockSpec((_DIN, _PDIM), lambda i: (0, 0)),
            pl.BlockSpec((_DIN, _PDIM), lambda i: (0, 0)),
            pl.BlockSpec((_DIN, _DIN), lambda i: (0, 0)),
            pl.BlockSpec((_DIN, _DIN), lambda i: (0, 0)),
            pl.BlockSpec((_PDIM, _DIN), lambda i: (0, 0)),
            pl.BlockSpec((1, _DIN), lambda i: (0, 0)),
            pl.BlockSpec((1, _DIN), lambda i: (0, 0)),
            pl.BlockSpec((1, _DIN), lambda i: (0, 0)),
            pl.BlockSpec((1, 1), lambda i: (0, 0)),
            pl.BlockSpec((_DIN, _HID), lambda i: (0, 0)),
            pl.BlockSpec((1, _HID), lambda i: (0, 0)),
            pl.BlockSpec((1, _HID), lambda i: (0, 0)),
            pl.BlockSpec((1, _HID), lambda i: (0, 0)),
        ],
        out_specs=pl.BlockSpec((blk, _HID), lambda i: (i, 0)),
        out_shape=jax.ShapeDtypeStruct((_N, _HID), jnp.float32),
    )(phi, Si, Sj, WcT[:_DIN], WcT[_DIN:2 * _DIN], WcT[2 * _DIN:],
      bc.reshape(1, _DIN), lnq_g.reshape(1, _DIN), lnq_b.reshape(1, _DIN),
      alpha.reshape(1, 1), Wlc.T, blc.reshape(1, _HID),
      lnc_g.reshape(1, _HID), lnc_b.reshape(1, _HID))


def _proj_body(h_ref, w1_ref, b1_ref, w2_ref, b2_ref, g_ref, b_ref,
               ts_ref, td_ref, hout_ref, *, pre_ln):
    hin = h_ref[...]
    if pre_ln:
        hin = _elu(_layer_norm(hin, g_ref[...], b_ref[...]))
        hout_ref[...] = hin
    h1 = jnp.dot(hin, w1_ref[...],
                 preferred_element_type=jnp.float32) + b1_ref[...]
    h2 = jnp.dot(hin, w2_ref[...],
                 preferred_element_type=jnp.float32) + b2_ref[...]
    hp = h1 * h2
    ts_ref[:, :_HD] = h1
    ts_ref[:, _HD:] = hp
    td_ref[:, :_HD] = h2
    td_ref[:, _HD:] = hp


def _layer_proj(hin, W1, b1, W2, b2, ln_g, ln_b, pre_ln):
    blk = 400
    k = hin.shape[1]
    outs = [
        jax.ShapeDtypeStruct((_N, 2 * _HD), jnp.float32),
        jax.ShapeDtypeStruct((_N, 2 * _HD), jnp.float32),
        jax.ShapeDtypeStruct((_N, k), jnp.float32),
    ]
    res = pl.pallas_call(
        functools.partial(_proj_body, pre_ln=pre_ln),
        grid=(_N // blk,),
        in_specs=[
            pl.BlockSpec((blk, k), lambda i: (i, 0)),
            pl.BlockSpec((k, _HD), lambda i: (0, 0)),
            pl.BlockSpec((1, _HD), lambda i: (0, 0)),
            pl.BlockSpec((k, _HD), lambda i: (0, 0)),
            pl.BlockSpec((1, _HD), lambda i: (0, 0)),
            pl.BlockSpec((1, k), lambda i: (0, 0)),
            pl.BlockSpec((1, k), lambda i: (0, 0)),
        ],
        out_specs=[
            pl.BlockSpec((blk, 2 * _HD), lambda i: (i, 0)),
            pl.BlockSpec((blk, 2 * _HD), lambda i: (i, 0)),
            pl.BlockSpec((blk, k), lambda i: (i, 0)),
        ],
        out_shape=outs,
    )(hin, W1.T, b1.reshape(1, _HD), W2.T, b2.reshape(1, _HD),
      ln_g.reshape(1, k), ln_b.reshape(1, k))
    return res[0], res[1], res[2]


def _final_body(o1_ref, r_ref, g_ref, b_ref, w_ref, bo_ref, y_ref):
    h = _elu(_layer_norm(o1_ref[...], g_ref[...], b_ref[...])) + r_ref[...]
    y_ref[...] = jnp.dot(h, w_ref[...],
                         preferred_element_type=jnp.float32) + bo_ref[...]


def _final_proj(out1, hres, ln_g, ln_b, Wo, bo):
    blk = 1000
    out_d = Wo.shape[0]
    return pl.pallas_call(
        _final_body,
        grid=(_N // blk,),
        in_specs=[
            pl.BlockSpec((blk, _HD), lambda i: (i, 0)),
            pl.BlockSpec((blk, _HD), lambda i: (i, 0)),
            pl.BlockSpec((1, _HD), lambda i: (0, 0)),
            pl.BlockSpec((1, _HD), lambda i: (0, 0)),
            pl.BlockSpec((_HD, out_d), lambda i: (0, 0)),
            pl.BlockSpec((1, out_d), lambda i: (0, 0)),
        ],
        out_specs=pl.BlockSpec((blk, out_d), lambda i: (i, 0)),
        out_shape=jax.ShapeDtypeStruct((_N, out_d), jnp.float32),
    )(out1, hres, ln_g.reshape(1, _HD), ln_b.reshape(1, _HD), Wo.T,
      bo.reshape(1, out_d))


def kernel(x, params, edge_index):
    p = params
    src = edge_index[0]
    dst = edge_index[1]
    zrows = jnp.zeros((10240 // 16, _DH), jnp.float32)

    phi, ssum, ssq = _front_stats(x, p['Wp'], p['bp'])
    zv = ((ssq - ssum * ssum / _N) / (_N - 1)).reshape(_DIN)
    _, topk = jax.lax.top_k(zv, _KP)
    ar = jnp.arange(_DIN, dtype=jnp.int32)
    Si = (ar[:, None] == topk[_PI][None, :]).astype(jnp.float32)
    Sj = (ar[:, None] == topk[_PJ][None, :]).astype(jnp.float32)
    xc = _front_feat(phi, Si, Sj, p['Wc'], p['bc'], p['lnq_g'], p['lnq_b'],
                     p['alpha'], p['Wlc'], p['blc'], p['lnc_g'], p['lnc_b'])

    Ts0, Td0, _unused = _layer_proj(xc, p['g0_W1'], p['g0_b1'],
                                    p['g0_W2'], p['g0_b2'],
                                    p['lnc_g'], p['lnc_b'], pre_ln=False)
    out0 = _gat_edge_stage(Ts0, Td0, src, dst, zrows, p['g0_a'])

    Ts1, Td1, hres = _layer_proj(out0, p['g1_W1'], p['g1_b1'],
                                 p['g1_W2'], p['g1_b2'],
                                 p['ln0_g'], p['ln0_b'], pre_ln=True)
    out1 = _gat_edge_stage(Ts1, Td1, src, dst, zrows, p['g1_a'])

    return _final_proj(out1, hres, p['ln1_g'], p['ln1_b'], p['Wo'], p['bo'])


# speedup vs baseline: 5.8849x; 1.1433x over previous
import functools

import numpy as np
import jax
import jax.numpy as jnp
from jax.experimental import pallas as pl
from jax.experimental.pallas import tpu as pltpu
from jax.experimental.pallas import tpu_sc as plsc

_N = 10000
_E = 160000
_DIN = 256
_HID = 128
_H = 8
_DH = 128
_HD = _H * _DH
_KP = 32
_PDIM = 32


def _mk_pairs(kp=_KP, pdim=_PDIM):
    ii, jj = [], []
    done = False
    for i in range(kp):
        for j in range(i + 1, kp):
            ii.append(i)
            jj.append(j)
            if len(ii) >= pdim:
                done = True
                break
        if done:
            break
    return np.array(ii), np.array(jj)


_PI, _PJ = _mk_pairs()


def _layer_norm(x, g, b, eps=1e-5):
    m = x.mean(-1, keepdims=True)
    v = ((x - m) ** 2).mean(-1, keepdims=True)
    return (x - m) / jnp.sqrt(v + eps) * g + b


def _sc_gather_call(Ts, Td, src, dst):
    info = plsc.get_sparse_core_info()
    NC, NS = info.num_cores, info.num_subcores
    NW = NC * NS
    per_w = _E // NW
    C = 8
    n_it = per_w // C
    W2 = 2 * _HD
    mesh = plsc.VectorSubcoreMesh(core_axis_name="c", subcore_axis_name="s")

    @functools.partial(
        pl.kernel,
        mesh=mesh,
        out_type=[
            jax.ShapeDtypeStruct((_E, W2), jnp.float32),
            jax.ShapeDtypeStruct((_E, W2), jnp.float32),
        ],
        scratch_types=[
            pltpu.VMEM((2, C), jnp.int32),
            pltpu.VMEM((2, C), jnp.int32),
            pltpu.VMEM((2, C, W2), jnp.float32),
            pltpu.VMEM((2, C, W2), jnp.float32),
            pltpu.SemaphoreType.DMA,
            pltpu.SemaphoreType.DMA,
        ],
    )
    def k(ts_hbm, td_hbm, src_hbm, dst_hbm, gs_hbm, gd_hbm,
          sidx, didx, bufs, bufd, sem0, sem1):
        wid = jax.lax.axis_index("s") * NC + jax.lax.axis_index("c")
        base = wid * per_w
        sems = (sem0, sem1)

        def fire(j, b):
            e0 = base + j * C
            pltpu.sync_copy(src_hbm.at[pl.ds(e0, C)], sidx.at[b])
            pltpu.sync_copy(dst_hbm.at[pl.ds(e0, C)], didx.at[b])
            pltpu.async_copy(ts_hbm.at[sidx.at[b]], bufs.at[b], sems[b])
            pltpu.async_copy(td_hbm.at[didx.at[b]], bufd.at[b], sems[b])

        def consume(j, b):
            pltpu.make_async_copy(ts_hbm.at[sidx.at[b]], bufs.at[b],
                                  sems[b]).wait()
            pltpu.make_async_copy(td_hbm.at[didx.at[b]], bufd.at[b],
                                  sems[b]).wait()
            e0 = base + j * C
            pltpu.sync_copy(bufs.at[b], gs_hbm.at[pl.ds(e0, C)])
            pltpu.sync_copy(bufd.at[b], gd_hbm.at[pl.ds(e0, C)])

        fire(0, 0)

        def body(i, carry):
            j0 = 2 * i
            fire(j0 + 1, 1)
            consume(j0, 0)

            @pl.when(j0 + 2 < n_it)
            def _():
                fire(j0 + 2, 0)

            consume(j0 + 1, 1)
            return carry

        jax.lax.fori_loop(0, n_it // 2, body, 0)
        if n_it % 2 == 1:
            consume(n_it - 1, 0)

    return k(Ts, Td, src, dst)


def _attn_msg_body(gs_ref, gd_ref, a_ref, msg_ref, *, eb):
    h1s = gs_ref[:, :_HD]
    hps = gs_ref[:, _HD:]
    h2d = gd_ref[:, :_HD]
    hpd = gd_ref[:, _HD:]
    t = h1s + h2d + hps * hpd
    t = jnp.where(t > 0, t, jnp.exp(t) - 1.0)
    w = (t * a_ref[...]).reshape(eb, _H, _DH)
    logits = jnp.sum(w, axis=-1) * np.float32(1.0 / np.sqrt(_DH))
    m = jnp.max(logits, axis=1, keepdims=True)
    ex = jnp.exp(logits - m)
    attn = ex / jnp.sum(ex, axis=1, keepdims=True)
    msg = attn[:, :, None] * h1s.reshape(eb, _H, _DH)
    msg_ref[...] = msg.reshape(eb, _HD)


def _attn_msg_call(Gs, Gd, a_flat):
    eb = 320
    W2 = 2 * _HD
    return pl.pallas_call(
        functools.partial(_attn_msg_body, eb=eb),
        grid=(_E // eb,),
        in_specs=[
            pl.BlockSpec((eb, W2), lambda i: (i, 0)),
            pl.BlockSpec((eb, W2), lambda i: (i, 0)),
            pl.BlockSpec((1, _HD), lambda i: (0, 0)),
        ],
        out_specs=pl.BlockSpec((eb, _HD), lambda i: (i, 0)),
        out_shape=jax.ShapeDtypeStruct((_E, _HD), jnp.float32),
    )(Gs, Gd, a_flat)


def _sc_scatter_call(msg, dst, zrows):
    info = plsc.get_sparse_core_info()
    NC, NS = info.num_cores, info.num_subcores
    per_t = _E // NS
    C2 = 80
    n_it = per_t // C2
    NPAD = 10240
    zpt = NPAD // NS
    opt = 624
    hpc = _H // NC
    mesh = plsc.VectorSubcoreMesh(core_axis_name="c", subcore_axis_name="s")

    @functools.partial(
        pl.kernel,
        mesh=mesh,
        out_type=jax.ShapeDtypeStruct((_N, _HD), jnp.float32),
        scratch_types=[
            pltpu.VMEM((C2,), jnp.int32),
            pltpu.VMEM((C2,), jnp.int32),
            pltpu.VMEM((C2, _DH), jnp.float32),
            pltpu.VMEM((C2, _DH), jnp.float32),
            pltpu.VMEM_SHARED((NPAD, _DH), jnp.float32),
            pltpu.SemaphoreType.DMA,
            pltpu.SemaphoreType.DMA,
        ],
    )
    def k(msg_hbm, dst_hbm, z_hbm, out_hbm, didx0, didx1, mbuf0, mbuf1,
          acc_sh, sem0, sem1):
        c = jax.lax.axis_index("c")
        s = jax.lax.axis_index("s")
        didxs = (didx0, didx1)
        mbufs = (mbuf0, mbuf1)
        sems = (sem0, sem1)
        for hh in range(hpc):
            h = c * hpc + hh
            col = h * _DH
            pltpu.sync_copy(z_hbm, acc_sh.at[pl.ds(s * zpt, zpt)])
            plsc.subcore_barrier()

            def fire(j, b):
                e0 = s * per_t + j * C2
                pltpu.async_copy(dst_hbm.at[pl.ds(e0, C2)], didxs[b],
                                 sems[b])
                pltpu.async_copy(
                    msg_hbm.at[pl.ds(e0, C2), pl.ds(col, _DH)],
                    mbufs[b], sems[b])

            def consume(j, b):
                e0 = s * per_t + j * C2
                pltpu.make_async_copy(dst_hbm.at[pl.ds(e0, C2)], didxs[b],
                                      sems[b]).wait()
                pltpu.make_async_copy(
                    msg_hbm.at[pl.ds(e0, C2), pl.ds(col, _DH)],
                    mbufs[b], sems[b]).wait()
                pltpu.sync_copy(mbufs[b], acc_sh.at[didxs[b]], add=True)

            fire(0, 0)

            def body(i, carry):
                j0 = 2 * i
                fire(j0 + 1, 1)
                consume(j0, 0)

                @pl.when(j0 + 2 < n_it)
                def _():
                    fire(j0 + 2, 0)

                consume(j0 + 1, 1)
                return carry

            jax.lax.fori_loop(0, n_it // 2, body, 0)
            if n_it % 2 == 1:
                consume(n_it - 1, 0)
            plsc.subcore_barrier()
            pltpu.sync_copy(
                acc_sh.at[pl.ds(s * opt, opt)],
                out_hbm.at[pl.ds(s * opt, opt), pl.ds(col, _DH)])

            @pl.when(s == NS - 1)
            def _():
                pltpu.sync_copy(
                    acc_sh.at[pl.ds(NS * opt, _N - NS * opt)],
                    out_hbm.at[pl.ds(NS * opt, _N - NS * opt),
                               pl.ds(col, _DH)])

            plsc.subcore_barrier()

    return k(msg, dst, zrows)


def _gat_edge_stage(Ts, Td, src, dst, zrows, a):
    Gs, Gd = _sc_gather_call(Ts, Td, src, dst)
    msg = _attn_msg_call(Gs, Gd, a.reshape(1, _HD))
    return _sc_scatter_call(msg, dst, zrows)


def _elu(v):
    return jnp.where(v > 0, v, jnp.exp(v) - 1.0)


def _front_stats_body(x_ref, w_ref, b_ref, phi_ref, s_ref, ss_ref):
    z = jnp.dot(x_ref[...], w_ref[...],
                preferred_element_type=jnp.float32) + b_ref[...]
    phi_ref[...] = jnp.tanh(z)

    @pl.when(pl.program_id(0) == 0)
    def _():
        s_ref[...] = jnp.zeros_like(s_ref)
        ss_ref[...] = jnp.zeros_like(ss_ref)

    s_ref[...] += jnp.sum(z, axis=0, keepdims=True)
    ss_ref[...] += jnp.sum(z * z, axis=0, keepdims=True)


def _front_stats(x, Wp, bp):
    blk = 2000
    return pl.pallas_call(
        _front_stats_body,
        grid=(_N // blk,),
        in_specs=[
            pl.BlockSpec((blk, _DIN), lambda i: (i, 0)),
            pl.BlockSpec((_DIN, _DIN), lambda i: (0, 0)),
            pl.BlockSpec((1, _DIN), lambda i: (0, 0)),
        ],
        out_specs=[
            pl.BlockSpec((blk, _DIN), lambda i: (i, 0)),
            pl.BlockSpec((1, _DIN), lambda i: (0, 0)),
            pl.BlockSpec((1, _DIN), lambda i: (0, 0)),
        ],
        out_shape=[
            jax.ShapeDtypeStruct((_N, _DIN), jnp.float32),
            jax.ShapeDtypeStruct((1, _DIN), jnp.float32),
            jax.ShapeDtypeStruct((1, _DIN), jnp.float32),
        ],
    )(x, Wp.T, bp.reshape(1, _DIN))


def _front_feat_body(phi_ref, si_ref, sj_ref, wa_ref, wb_ref, wc_ref,
                     bc_ref, lg_ref, lb_ref, al_ref, wl_ref, blc_ref,
                     cg_ref, cb_ref, xc_ref):
    phi = phi_ref[...]
    qc = jnp.cos(np.float32(np.pi) * phi)
    qs = jnp.sin(np.float32(np.pi) * phi)
    phi_i = jnp.dot(phi, si_ref[...], preferred_element_type=jnp.float32)
    phi_j = jnp.dot(phi, sj_ref[...], preferred_element_type=jnp.float32)
    qp = jnp.cos(phi_i - phi_j)
    Q = (jnp.dot(qc, wa_ref[...], preferred_element_type=jnp.float32)
         + jnp.dot(qs, wb_ref[...], preferred_element_type=jnp.float32)
         + jnp.dot(qp, wc_ref[...], preferred_element_type=jnp.float32)
         + bc_ref[...])
    Q = _layer_norm(Q, lg_ref[...], lb_ref[...]) * al_ref[0, 0]
    xc = jnp.dot(Q, wl_ref[...],
                 preferred_element_type=jnp.float32) + blc_ref[...]
    xc_ref[...] = _elu(_layer_norm(xc, cg_ref[...], cb_ref[...]))


def _front_feat(phi, Si, Sj, Wc, bc, lnq_g, lnq_b, alpha, Wlc, blc,
                lnc_g, lnc_b):
    blk = 2000
    WcT = Wc.T
    return pl.pallas_call(
        _front_feat_body,
        grid=(_N // blk,),
        in_specs=[
            pl.BlockSpec((blk, _DIN), lambda i: (i, 0)),
            pl.BlockSpec((_DIN, _PDIM), lambda i: (0, 0)),
            pl.BlockSpec((_DIN, _PDIM), lambda i: (0, 0)),
            pl.BlockSpec((_DIN, _DIN), lambda i: (0, 0)),
            pl.BlockSpec((_DIN, _DIN), lambda i: (0, 0)),
            pl.BlockSpec((_PDIM, _DIN), lambda i: (0, 0)),
            pl.BlockSpec((1, _DIN), lambda i: (0, 0)),
            pl.BlockSpec((1, _DIN), lambda i: (0, 0)),
            pl.BlockSpec((1, _DIN), lambda i: (0, 0)),
            pl.BlockSpec((1, 1), lambda i: (0, 0)),
            pl.BlockSpec((_DIN, _HID), lambda i: (0, 0)),
            pl.BlockSpec((1, _HID), lambda i: (0, 0)),
            pl.BlockSpec((1, _HID), lambda i: (0, 0)),
            pl.BlockSpec((1, _HID), lambda i: (0, 0)),
        ],
        out_specs=pl.BlockSpec((blk, _HID), lambda i: (i, 0)),
        out_shape=jax.ShapeDtypeStruct((_N, _HID), jnp.float32),
    )(phi, Si, Sj, WcT[:_DIN], WcT[_DIN:2 * _DIN], WcT[2 * _DIN:],
      bc.reshape(1, _DIN), lnq_g.reshape(1, _DIN), lnq_b.reshape(1, _DIN),
      alpha.reshape(1, 1), Wlc.T, blc.reshape(1, _HID),
      lnc_g.reshape(1, _HID), lnc_b.reshape(1, _HID))


def _proj_body(h_ref, w1_ref, b1_ref, w2_ref, b2_ref, g_ref, b_ref,
               ts_ref, td_ref, hout_ref, *, pre_ln):
    hin = h_ref[...]
    if pre_ln:
        hin = _elu(_layer_norm(hin, g_ref[...], b_ref[...]))
        hout_ref[...] = hin
    h1 = jnp.dot(hin, w1_ref[...],
                 preferred_element_type=jnp.float32) + b1_ref[...]
    h2 = jnp.dot(hin, w2_ref[...],
                 preferred_element_type=jnp.float32) + b2_ref[...]
    hp = h1 * h2
    ts_ref[:, :_HD] = h1
    ts_ref[:, _HD:] = hp
    td_ref[:, :_HD] = h2
    td_ref[:, _HD:] = hp


def _layer_proj(hin, W1, b1, W2, b2, ln_g, ln_b, pre_ln):
    blk = 400
    k = hin.shape[1]
    outs = [
        jax.ShapeDtypeStruct((_N, 2 * _HD), jnp.float32),
        jax.ShapeDtypeStruct((_N, 2 * _HD), jnp.float32),
        jax.ShapeDtypeStruct((_N, k), jnp.float32),
    ]
    res = pl.pallas_call(
        functools.partial(_proj_body, pre_ln=pre_ln),
        grid=(_N // blk,),
        in_specs=[
            pl.BlockSpec((blk, k), lambda i: (i, 0)),
            pl.BlockSpec((k, _HD), lambda i: (0, 0)),
            pl.BlockSpec((1, _HD), lambda i: (0, 0)),
            pl.BlockSpec((k, _HD), lambda i: (0, 0)),
            pl.BlockSpec((1, _HD), lambda i: (0, 0)),
            pl.BlockSpec((1, k), lambda i: (0, 0)),
            pl.BlockSpec((1, k), lambda i: (0, 0)),
        ],
        out_specs=[
            pl.BlockSpec((blk, 2 * _HD), lambda i: (i, 0)),
            pl.BlockSpec((blk, 2 * _HD), lambda i: (i, 0)),
            pl.BlockSpec((blk, k), lambda i: (i, 0)),
        ],
        out_shape=outs,
    )(hin, W1.T, b1.reshape(1, _HD), W2.T, b2.reshape(1, _HD),
      ln_g.reshape(1, k), ln_b.reshape(1, k))
    return res[0], res[1], res[2]


def _final_body(o1_ref, r_ref, g_ref, b_ref, w_ref, bo_ref, y_ref):
    h = _elu(_layer_norm(o1_ref[...], g_ref[...], b_ref[...])) + r_ref[...]
    y_ref[...] = jnp.dot(h, w_ref[...],
                         preferred_element_type=jnp.float32) + bo_ref[...]


def _final_proj(out1, hres, ln_g, ln_b, Wo, bo):
    blk = 1000
    out_d = Wo.shape[0]
    return pl.pallas_call(
        _final_body,
        grid=(_N // blk,),
        in_specs=[
            pl.BlockSpec((blk, _HD), lambda i: (i, 0)),
            pl.BlockSpec((blk, _HD), lambda i: (i, 0)),
            pl.BlockSpec((1, _HD), lambda i: (0, 0)),
            pl.BlockSpec((1, _HD), lambda i: (0, 0)),
            pl.BlockSpec((_HD, out_d), lambda i: (0, 0)),
            pl.BlockSpec((1, out_d), lambda i: (0, 0)),
        ],
        out_specs=pl.BlockSpec((blk, out_d), lambda i: (i, 0)),
        out_shape=jax.ShapeDtypeStruct((_N, out_d), jnp.float32),
    )(out1, hres, ln_g.reshape(1, _HD), ln_b.reshape(1, _HD), Wo.T,
      bo.reshape(1, out_d))


def kernel(x, params, edge_index):
    p = params
    src = edge_index[0]
    dst = edge_index[1]
    zrows = jnp.zeros((10240 // 16, _DH), jnp.float32)

    phi, ssum, ssq = _front_stats(x, p['Wp'], p['bp'])
    zv = ((ssq - ssum * ssum / _N) / (_N - 1)).reshape(_DIN)
    _, topk = jax.lax.top_k(zv, _KP)
    ar = jnp.arange(_DIN, dtype=jnp.int32)
    Si = (ar[:, None] == topk[_PI][None, :]).astype(jnp.float32)
    Sj = (ar[:, None] == topk[_PJ][None, :]).astype(jnp.float32)
    xc = _front_feat(phi, Si, Sj, p['Wc'], p['bc'], p['lnq_g'], p['lnq_b'],
                     p['alpha'], p['Wlc'], p['blc'], p['lnc_g'], p['lnc_b'])

    Ts0, Td0, _unused = _layer_proj(xc, p['g0_W1'], p['g0_b1'],
                                    p['g0_W2'], p['g0_b2'],
                                    p['lnc_g'], p['lnc_b'], pre_ln=False)
    out0 = _gat_edge_stage(Ts0, Td0, src, dst, zrows, p['g0_a'])

    Ts1, Td1, hres = _layer_proj(out0, p['g1_W1'], p['g1_b1'],
                                 p['g1_W2'], p['g1_b2'],
                                 p['ln0_g'], p['ln0_b'], pre_ln=True)
    out1 = _gat_edge_stage(Ts1, Td1, src, dst, zrows, p['g1_a'])

    return _final_proj(out1, hres, p['ln1_g'], p['ln1_b'], p['Wo'], p['bo'])


# speedup vs baseline: 6.1567x; 1.0462x over previous
import functools

import numpy as np
import jax
import jax.numpy as jnp
from jax.experimental import pallas as pl
from jax.experimental.pallas import tpu as pltpu
from jax.experimental.pallas import tpu_sc as plsc

_N = 10000
_E = 160000
_DIN = 256
_HID = 128
_H = 8
_DH = 128
_HD = _H * _DH
_KP = 32
_PDIM = 32


def _mk_pairs(kp=_KP, pdim=_PDIM):
    ii, jj = [], []
    done = False
    for i in range(kp):
        for j in range(i + 1, kp):
            ii.append(i)
            jj.append(j)
            if len(ii) >= pdim:
                done = True
                break
        if done:
            break
    return np.array(ii), np.array(jj)


_PI, _PJ = _mk_pairs()


def _layer_norm(x, g, b, eps=1e-5):
    m = x.mean(-1, keepdims=True)
    v = ((x - m) ** 2).mean(-1, keepdims=True)
    return (x - m) / jnp.sqrt(v + eps) * g + b


def _sc_gather_call(Ts, Td, src, dst):
    info = plsc.get_sparse_core_info()
    NC, NS = info.num_cores, info.num_subcores
    NW = NC * NS
    per_w = _E // NW
    C = 8
    n_it = per_w // C
    W2 = 2 * _HD
    mesh = plsc.VectorSubcoreMesh(core_axis_name="c", subcore_axis_name="s")

    @functools.partial(
        pl.kernel,
        mesh=mesh,
        out_type=[
            jax.ShapeDtypeStruct((_E, W2), jnp.float32),
            jax.ShapeDtypeStruct((_E, W2), jnp.float32),
        ],
        scratch_types=[
            pltpu.VMEM((2, C), jnp.int32),
            pltpu.VMEM((2, C), jnp.int32),
            pltpu.VMEM((2, C, W2), jnp.float32),
            pltpu.VMEM((2, C, W2), jnp.float32),
            pltpu.SemaphoreType.DMA,
            pltpu.SemaphoreType.DMA,
        ],
    )
    def k(ts_hbm, td_hbm, src_hbm, dst_hbm, gs_hbm, gd_hbm,
          sidx, didx, bufs, bufd, sem0, sem1):
        wid = jax.lax.axis_index("s") * NC + jax.lax.axis_index("c")
        base = wid * per_w
        sems = (sem0, sem1)

        def fire(j, b):
            e0 = base + j * C
            pltpu.sync_copy(src_hbm.at[pl.ds(e0, C)], sidx.at[b])
            pltpu.sync_copy(dst_hbm.at[pl.ds(e0, C)], didx.at[b])
            pltpu.async_copy(ts_hbm.at[sidx.at[b]], bufs.at[b], sems[b])
            pltpu.async_copy(td_hbm.at[didx.at[b]], bufd.at[b], sems[b])

        def consume(j, b):
            pltpu.make_async_copy(ts_hbm.at[sidx.at[b]], bufs.at[b],
                                  sems[b]).wait()
            pltpu.make_async_copy(td_hbm.at[didx.at[b]], bufd.at[b],
                                  sems[b]).wait()
            e0 = base + j * C
            pltpu.sync_copy(bufs.at[b], gs_hbm.at[pl.ds(e0, C)])
            pltpu.sync_copy(bufd.at[b], gd_hbm.at[pl.ds(e0, C)])

        fire(0, 0)

        def body(i, carry):
            j0 = 2 * i
            fire(j0 + 1, 1)
            consume(j0, 0)

            @pl.when(j0 + 2 < n_it)
            def _():
                fire(j0 + 2, 0)

            consume(j0 + 1, 1)
            return carry

        jax.lax.fori_loop(0, n_it // 2, body, 0)
        if n_it % 2 == 1:
            consume(n_it - 1, 0)

    return k(Ts, Td, src, dst)


def _attn_msg_body(gs_ref, gd_ref, a_ref, msg_ref, *, eb):
    h1s = gs_ref[:, :_HD]
    hps = gs_ref[:, _HD:]
    h2d = gd_ref[:, :_HD]
    hpd = gd_ref[:, _HD:]
    t = h1s + h2d + hps * hpd
    t = jnp.where(t > 0, t, jnp.exp(t) - 1.0)
    w = (t * a_ref[...]).reshape(eb, _H, _DH)
    logits = jnp.sum(w, axis=-1) * np.float32(1.0 / np.sqrt(_DH))
    m = jnp.max(logits, axis=1, keepdims=True)
    ex = jnp.exp(logits - m)
    attn = ex / jnp.sum(ex, axis=1, keepdims=True)
    msg = attn[:, :, None] * h1s.reshape(eb, _H, _DH)
    msg_ref[...] = msg.reshape(eb, _HD)


def _attn_msg_call(Gs, Gd, a_flat):
    eb = 640
    W2 = 2 * _HD
    return pl.pallas_call(
        functools.partial(_attn_msg_body, eb=eb),
        grid=(_E // eb,),
        in_specs=[
            pl.BlockSpec((eb, W2), lambda i: (i, 0)),
            pl.BlockSpec((eb, W2), lambda i: (i, 0)),
            pl.BlockSpec((1, _HD), lambda i: (0, 0)),
        ],
        out_specs=pl.BlockSpec((eb, _HD), lambda i: (i, 0)),
        out_shape=jax.ShapeDtypeStruct((_E, _HD), jnp.float32),
    )(Gs, Gd, a_flat)


def _sc_scatter_call(msg, dst, zrows):
    info = plsc.get_sparse_core_info()
    NC, NS = info.num_cores, info.num_subcores
    per_t = _E // NS
    C2 = 80
    n_it = per_t // C2
    NPAD = 10240
    zpt = NPAD // NS
    opt = 624
    hpc = _H // NC
    mesh = plsc.VectorSubcoreMesh(core_axis_name="c", subcore_axis_name="s")

    @functools.partial(
        pl.kernel,
        mesh=mesh,
        out_type=jax.ShapeDtypeStruct((_N, _HD), jnp.float32),
        scratch_types=[
            pltpu.VMEM((C2,), jnp.int32),
            pltpu.VMEM((C2,), jnp.int32),
            pltpu.VMEM((C2, _DH), jnp.float32),
            pltpu.VMEM((C2, _DH), jnp.float32),
            pltpu.VMEM_SHARED((NPAD, _DH), jnp.float32),
            pltpu.SemaphoreType.DMA,
            pltpu.SemaphoreType.DMA,
        ],
    )
    def k(msg_hbm, dst_hbm, z_hbm, out_hbm, didx0, didx1, mbuf0, mbuf1,
          acc_sh, sem0, sem1):
        c = jax.lax.axis_index("c")
        s = jax.lax.axis_index("s")
        didxs = (didx0, didx1)
        mbufs = (mbuf0, mbuf1)
        sems = (sem0, sem1)
        for hh in range(hpc):
            h = c * hpc + hh
            col = h * _DH
            pltpu.sync_copy(z_hbm, acc_sh.at[pl.ds(s * zpt, zpt)])
            plsc.subcore_barrier()

            def fire(j, b):
                e0 = s * per_t + j * C2
                pltpu.async_copy(dst_hbm.at[pl.ds(e0, C2)], didxs[b],
                                 sems[b])
                pltpu.async_copy(
                    msg_hbm.at[pl.ds(e0, C2), pl.ds(col, _DH)],
                    mbufs[b], sems[b])

            def consume(j, b):
                e0 = s * per_t + j * C2
                pltpu.make_async_copy(dst_hbm.at[pl.ds(e0, C2)], didxs[b],
                                      sems[b]).wait()
                pltpu.make_async_copy(
                    msg_hbm.at[pl.ds(e0, C2), pl.ds(col, _DH)],
                    mbufs[b], sems[b]).wait()
                pltpu.sync_copy(mbufs[b], acc_sh.at[didxs[b]], add=True)

            fire(0, 0)

            def body(i, carry):
                j0 = 2 * i
                fire(j0 + 1, 1)
                consume(j0, 0)

                @pl.when(j0 + 2 < n_it)
                def _():
                    fire(j0 + 2, 0)

                consume(j0 + 1, 1)
                return carry

            jax.lax.fori_loop(0, n_it // 2, body, 0)
            if n_it % 2 == 1:
                consume(n_it - 1, 0)
            plsc.subcore_barrier()
            pltpu.sync_copy(
                acc_sh.at[pl.ds(s * opt, opt)],
                out_hbm.at[pl.ds(s * opt, opt), pl.ds(col, _DH)])

            @pl.when(s == NS - 1)
            def _():
                pltpu.sync_copy(
                    acc_sh.at[pl.ds(NS * opt, _N - NS * opt)],
                    out_hbm.at[pl.ds(NS * opt, _N - NS * opt),
                               pl.ds(col, _DH)])

            plsc.subcore_barrier()

    return k(msg, dst, zrows)


def _gat_edge_stage(Ts, Td, src, dst, zrows, a):
    Gs, Gd = _sc_gather_call(Ts, Td, src, dst)
    msg = _attn_msg_call(Gs, Gd, a.reshape(1, _HD))
    return _sc_scatter_call(msg, dst, zrows)


def _elu(v):
    return jnp.where(v > 0, v, jnp.exp(v) - 1.0)


def _front_stats_body(x_ref, w_ref, b_ref, phi_ref, s_ref, ss_ref):
    z = jnp.dot(x_ref[...], w_ref[...],
                preferred_element_type=jnp.float32) + b_ref[...]
    phi_ref[...] = jnp.tanh(z)

    @pl.when(pl.program_id(0) == 0)
    def _():
        s_ref[...] = jnp.zeros_like(s_ref)
        ss_ref[...] = jnp.zeros_like(ss_ref)

    s_ref[...] += jnp.sum(z, axis=0, keepdims=True)
    ss_ref[...] += jnp.sum(z * z, axis=0, keepdims=True)


def _front_stats(x, Wp, bp):
    blk = 2000
    return pl.pallas_call(
        _front_stats_body,
        grid=(_N // blk,),
        in_specs=[
            pl.BlockSpec((blk, _DIN), lambda i: (i, 0)),
            pl.BlockSpec((_DIN, _DIN), lambda i: (0, 0)),
            pl.BlockSpec((1, _DIN), lambda i: (0, 0)),
        ],
        out_specs=[
            pl.BlockSpec((blk, _DIN), lambda i: (i, 0)),
            pl.BlockSpec((1, _DIN), lambda i: (0, 0)),
            pl.BlockSpec((1, _DIN), lambda i: (0, 0)),
        ],
        out_shape=[
            jax.ShapeDtypeStruct((_N, _DIN), jnp.float32),
            jax.ShapeDtypeStruct((1, _DIN), jnp.float32),
            jax.ShapeDtypeStruct((1, _DIN), jnp.float32),
        ],
    )(x, Wp.T, bp.reshape(1, _DIN))


def _front_feat_body(phi_ref, si_ref, sj_ref, wa_ref, wb_ref, wc_ref,
                     bc_ref, lg_ref, lb_ref, al_ref, wl_ref, blc_ref,
                     cg_ref, cb_ref, xc_ref):
    phi = phi_ref[...]
    qc = jnp.cos(np.float32(np.pi) * phi)
    qs = jnp.sin(np.float32(np.pi) * phi)
    phi_i = jnp.dot(phi, si_ref[...], preferred_element_type=jnp.float32)
    phi_j = jnp.dot(phi, sj_ref[...], preferred_element_type=jnp.float32)
    qp = jnp.cos(phi_i - phi_j)
    Q = (jnp.dot(qc, wa_ref[...], preferred_element_type=jnp.float32)
         + jnp.dot(qs, wb_ref[...], preferred_element_type=jnp.float32)
         + jnp.dot(qp, wc_ref[...], preferred_element_type=jnp.float32)
         + bc_ref[...])
    Q = _layer_norm(Q, lg_ref[...], lb_ref[...]) * al_ref[0, 0]
    xc = jnp.dot(Q, wl_ref[...],
                 preferred_element_type=jnp.float32) + blc_ref[...]
    xc_ref[...] = _elu(_layer_norm(xc, cg_ref[...], cb_ref[...]))


def _front_feat(phi, Si, Sj, Wc, bc, lnq_g, lnq_b, alpha, Wlc, blc,
                lnc_g, lnc_b):
    blk = 2000
    WcT = Wc.T
    return pl.pallas_call(
        _front_feat_body,
        grid=(_N // blk,),
        in_specs=[
            pl.BlockSpec((blk, _DIN), lambda i: (i, 0)),
            pl.BlockSpec((_DIN, _PDIM), lambda i: (0, 0)),
            pl.BlockSpec((_DIN, _PDIM), lambda i: (0, 0)),
            pl.BlockSpec((_DIN, _DIN), lambda i: (0, 0)),
            pl.BlockSpec((_DIN, _DIN), lambda i: (0, 0)),
            pl.BlockSpec((_PDIM, _DIN), lambda i: (0, 0)),
            pl.BlockSpec((1, _DIN), lambda i: (0, 0)),
            pl.BlockSpec((1, _DIN), lambda i: (0, 0)),
            pl.BlockSpec((1, _DIN), lambda i: (0, 0)),
            pl.BlockSpec((1, 1), lambda i: (0, 0)),
            pl.BlockSpec((_DIN, _HID), lambda i: (0, 0)),
            pl.BlockSpec((1, _HID), lambda i: (0, 0)),
            pl.BlockSpec((1, _HID), lambda i: (0, 0)),
            pl.BlockSpec((1, _HID), lambda i: (0, 0)),
        ],
        out_specs=pl.BlockSpec((blk, _HID), lambda i: (i, 0)),
        out_shape=jax.ShapeDtypeStruct((_N, _HID), jnp.float32),
    )(phi, Si, Sj, WcT[:_DIN], WcT[_DIN:2 * _DIN], WcT[2 * _DIN:],
      bc.reshape(1, _DIN), lnq_g.reshape(1, _DIN), lnq_b.reshape(1, _DIN),
      alpha.reshape(1, 1), Wlc.T, blc.reshape(1, _HID),
      lnc_g.reshape(1, _HID), lnc_b.reshape(1, _HID))


def _proj_body(h_ref, w1_ref, b1_ref, w2_ref, b2_ref, g_ref, b_ref,
               ts_ref, td_ref, hout_ref, *, pre_ln):
    hin = h_ref[...]
    if pre_ln:
        hin = _elu(_layer_norm(hin, g_ref[...], b_ref[...]))
        hout_ref[...] = hin
    h1 = jnp.dot(hin, w1_ref[...],
                 preferred_element_type=jnp.float32) + b1_ref[...]
    h2 = jnp.dot(hin, w2_ref[...],
                 preferred_element_type=jnp.float32) + b2_ref[...]
    hp = h1 * h2
    ts_ref[:, :_HD] = h1
    ts_ref[:, _HD:] = hp
    td_ref[:, :_HD] = h2
    td_ref[:, _HD:] = hp


def _layer_proj(hin, W1, b1, W2, b2, ln_g, ln_b, pre_ln):
    blk = 400
    k = hin.shape[1]
    outs = [
        jax.ShapeDtypeStruct((_N, 2 * _HD), jnp.float32),
        jax.ShapeDtypeStruct((_N, 2 * _HD), jnp.float32),
        jax.ShapeDtypeStruct((_N, k), jnp.float32),
    ]
    res = pl.pallas_call(
        functools.partial(_proj_body, pre_ln=pre_ln),
        grid=(_N // blk,),
        in_specs=[
            pl.BlockSpec((blk, k), lambda i: (i, 0)),
            pl.BlockSpec((k, _HD), lambda i: (0, 0)),
            pl.BlockSpec((1, _HD), lambda i: (0, 0)),
            pl.BlockSpec((k, _HD), lambda i: (0, 0)),
            pl.BlockSpec((1, _HD), lambda i: (0, 0)),
            pl.BlockSpec((1, k), lambda i: (0, 0)),
            pl.BlockSpec((1, k), lambda i: (0, 0)),
        ],
        out_specs=[
            pl.BlockSpec((blk, 2 * _HD), lambda i: (i, 0)),
            pl.BlockSpec((blk, 2 * _HD), lambda i: (i, 0)),
            pl.BlockSpec((blk, k), lambda i: (i, 0)),
        ],
        out_shape=outs,
    )(hin, W1.T, b1.reshape(1, _HD), W2.T, b2.reshape(1, _HD),
      ln_g.reshape(1, k), ln_b.reshape(1, k))
    return res[0], res[1], res[2]


def _final_body(o1_ref, r_ref, g_ref, b_ref, w_ref, bo_ref, y_ref):
    h = _elu(_layer_norm(o1_ref[...], g_ref[...], b_ref[...])) + r_ref[...]
    y_ref[...] = jnp.dot(h, w_ref[...],
                         preferred_element_type=jnp.float32) + bo_ref[...]


def _final_proj(out1, hres, ln_g, ln_b, Wo, bo):
    blk = 1000
    out_d = Wo.shape[0]
    return pl.pallas_call(
        _final_body,
        grid=(_N // blk,),
        in_specs=[
            pl.BlockSpec((blk, _HD), lambda i: (i, 0)),
            pl.BlockSpec((blk, _HD), lambda i: (i, 0)),
            pl.BlockSpec((1, _HD), lambda i: (0, 0)),
            pl.BlockSpec((1, _HD), lambda i: (0, 0)),
            pl.BlockSpec((_HD, out_d), lambda i: (0, 0)),
            pl.BlockSpec((1, out_d), lambda i: (0, 0)),
        ],
        out_specs=pl.BlockSpec((blk, out_d), lambda i: (i, 0)),
        out_shape=jax.ShapeDtypeStruct((_N, out_d), jnp.float32),
    )(out1, hres, ln_g.reshape(1, _HD), ln_b.reshape(1, _HD), Wo.T,
      bo.reshape(1, out_d))


def kernel(x, params, edge_index):
    p = params
    src = edge_index[0]
    dst = edge_index[1]
    zrows = jnp.zeros((10240 // 16, _DH), jnp.float32)

    phi, ssum, ssq = _front_stats(x, p['Wp'], p['bp'])
    zv = ((ssq - ssum * ssum / _N) / (_N - 1)).reshape(_DIN)
    _, topk = jax.lax.top_k(zv, _KP)
    ar = jnp.arange(_DIN, dtype=jnp.int32)
    Si = (ar[:, None] == topk[_PI][None, :]).astype(jnp.float32)
    Sj = (ar[:, None] == topk[_PJ][None, :]).astype(jnp.float32)
    xc = _front_feat(phi, Si, Sj, p['Wc'], p['bc'], p['lnq_g'], p['lnq_b'],
                     p['alpha'], p['Wlc'], p['blc'], p['lnc_g'], p['lnc_b'])

    Ts0, Td0, _unused = _layer_proj(xc, p['g0_W1'], p['g0_b1'],
                                    p['g0_W2'], p['g0_b2'],
                                    p['lnc_g'], p['lnc_b'], pre_ln=False)
    out0 = _gat_edge_stage(Ts0, Td0, src, dst, zrows, p['g0_a'])

    Ts1, Td1, hres = _layer_proj(out0, p['g1_W1'], p['g1_b1'],
                                 p['g1_W2'], p['g1_b2'],
                                 p['ln0_g'], p['ln0_b'], pre_ln=True)
    out1 = _gat_edge_stage(Ts1, Td1, src, dst, zrows, p['g1_a'])

    return _final_proj(out1, hres, p['ln1_g'], p['ln1_b'], p['Wo'], p['bo'])
